# Initial kernel scaffold; baseline (speedup 1.0000x reference)
import functools

import jax
import jax.numpy as jnp
from jax import lax
from jax.experimental import pallas as pl
from jax.experimental.pallas import tpu as pltpu
from jax.experimental.pallas import tpu_sc as plsc

N_NODES = 100000
N_EDGES = 1600000
D = 32
L = 16
NC = 2
NS = 16
NW = NC * NS
SUB = 128
NSUB = 4
CHUNK = SUB * NSUB
NCHUNKS = N_EDGES // CHUNK
BASE_CHUNKS = NCHUNKS // NW
EXTRA = NCHUNKS - BASE_CHUNKS * NW
DUMP = 6256
DUMP_LAST = N_NODES - 15 * DUMP

_mesh = plsc.VectorSubcoreMesh(core_axis_name="c", subcore_axis_name="s")


def _edge_body(src2_h, dst2_h, eu_h, ev_h, g2b_h, zeros_h,
               ag_h, psum_h,
               src_v, dst_v, u_v, v_v, ag_v, g2_v, sums_s,
               sem_u, sem_v):
    cid = lax.axis_index("c")
    sid = lax.axis_index("s")
    wid = sid * NC + cid

    @pl.when(sid == 0)
    def _():
        pltpu.sync_copy(zeros_h, sums_s)

    pltpu.sync_copy(g2b_h, g2_v)
    plsc.subcore_barrier()

    nmine = jnp.where(wid < EXTRA, BASE_CHUNKS + 1, BASE_CHUNKS)

    def chunk_body(k, carry):
        chunk = wid + k * NW
        row0 = chunk * NSUB
        ebase = chunk * CHUNK
        pltpu.sync_copy(src2_h.at[pl.ds(row0, NSUB)], src_v)
        pltpu.sync_copy(dst2_h.at[pl.ds(row0, NSUB)], dst_v)
        for j in range(NSUB):
            cu = pltpu.async_copy(eu_h.at[src_v.at[j]], u_v, sem_u)
            cv = pltpu.async_copy(ev_h.at[dst_v.at[j]], v_v, sem_v)
            cu.wait()
            cv.wait()

            def grp(kg, c2):
                eidx = kg * L + lax.iota(jnp.int32, L)
                acc = jnp.zeros((L,), jnp.float32)
                for d in range(D):
                    dsp = jnp.full((L,), d, jnp.int32)
                    ud = plsc.load_gather(u_v, [eidx, dsp])
                    vd = plsc.load_gather(v_v, [eidx, dsp])
                    acc = acc + ud * vd * g2_v[d]
                t = jnp.exp(acc)
                ag = 1.0 / (1.0 + jnp.exp(-t))
                ag_v[pl.ds(kg * L, L)] = ag
                return c2

            lax.fori_loop(0, SUB // L, grp, 0)
            pltpu.sync_copy(ag_v, sums_s.at[src_v.at[j]], add=True)
            pltpu.sync_copy(ag_v, ag_h.at[pl.ds(ebase + j * SUB, SUB)])
        return carry

    lax.fori_loop(0, nmine, chunk_body, 0)
    plsc.subcore_barrier()

    @pl.when(sid < NS - 1)
    def _():
        pltpu.sync_copy(sums_s.at[pl.ds(sid * DUMP, DUMP)],
                        psum_h.at[cid, pl.ds(sid * DUMP, DUMP)])

    @pl.when(sid == NS - 1)
    def _():
        pltpu.sync_copy(sums_s.at[pl.ds((NS - 1) * DUMP, DUMP_LAST)],
                        psum_h.at[cid, pl.ds((NS - 1) * DUMP, DUMP_LAST)])


_edge_kernel = functools.partial(
    pl.kernel,
    out_type=(
        jax.ShapeDtypeStruct((N_EDGES,), jnp.float32),
        jax.ShapeDtypeStruct((NC, N_NODES), jnp.float32),
    ),
    mesh=_mesh,
    scratch_types=[
        pltpu.VMEM((NSUB, SUB), jnp.int32),
        pltpu.VMEM((NSUB, SUB), jnp.int32),
        pltpu.VMEM((SUB, D), jnp.float32),
        pltpu.VMEM((SUB, D), jnp.float32),
        pltpu.VMEM((SUB,), jnp.float32),
        pltpu.VMEM((D, L), jnp.float32),
        pltpu.VMEM_SHARED((N_NODES,), jnp.float32),
        pltpu.SemaphoreType.DMA,
        pltpu.SemaphoreType.DMA,
    ],
)(_edge_body)


def _norm_body(src2_h, ag_h, p0_h, p1_h,
               c_h,
               src_v, s0_v, s1_v, ag_v, c_v,
               sem0, sem1):
    cid = lax.axis_index("c")
    sid = lax.axis_index("s")
    wid = sid * NC + cid
    nmine = jnp.where(wid < EXTRA, BASE_CHUNKS + 1, BASE_CHUNKS)

    def chunk_body(k, carry):
        chunk = wid + k * NW
        row0 = chunk * NSUB
        ebase = chunk * CHUNK
        pltpu.sync_copy(src2_h.at[pl.ds(row0, NSUB)], src_v)
        for j in range(NSUB):
            c0 = pltpu.async_copy(p0_h.at[src_v.at[j]], s0_v, sem0)
            c1 = pltpu.async_copy(p1_h.at[src_v.at[j]], s1_v, sem1)
            pltpu.sync_copy(ag_h.at[pl.ds(ebase + j * SUB, SUB)], ag_v)
            c0.wait()
            c1.wait()

            def grp(kg, c2):
                sl = pl.ds(kg * L, L)
                ag = ag_v[sl]
                s = s0_v[sl] + s1_v[sl]
                c = jnp.minimum(jnp.maximum(ag * 5.0 / s, 0.0), 1.0)
                c_v[sl] = c
                return c2

            lax.fori_loop(0, SUB // L, grp, 0)
            pltpu.sync_copy(c_v, c_h.at[pl.ds(ebase + j * SUB, SUB)])
        return carry

    lax.fori_loop(0, nmine, chunk_body, 0)


_norm_kernel = functools.partial(
    pl.kernel,
    out_type=jax.ShapeDtypeStruct((N_EDGES,), jnp.float32),
    mesh=_mesh,
    scratch_types=[
        pltpu.VMEM((NSUB, SUB), jnp.int32),
        pltpu.VMEM((SUB,), jnp.float32),
        pltpu.VMEM((SUB,), jnp.float32),
        pltpu.VMEM((SUB,), jnp.float32),
        pltpu.VMEM((SUB,), jnp.float32),
        pltpu.SemaphoreType.DMA,
        pltpu.SemaphoreType.DMA,
    ],
)(_norm_body)


def kernel(Eu, Ev, edge_index, g):
    src = edge_index[0]
    dst = edge_index[1]
    src2 = src.reshape(N_EDGES // SUB, SUB)
    dst2 = dst.reshape(N_EDGES // SUB, SUB)
    g2 = (g * g).reshape(D)
    g2b = jnp.broadcast_to(g2[:, None], (D, L))
    zeros = jnp.zeros((N_NODES,), jnp.float32)
    ag, psum = _edge_kernel(src2, dst2, Eu, Ev, g2b, zeros)
    return _norm_kernel(src2, ag, psum[0], psum[1])


# speedup vs baseline: 8.0555x; 8.0555x over previous
import functools

import jax
import jax.numpy as jnp
from jax import lax
from jax.experimental import pallas as pl
from jax.experimental.pallas import tpu as pltpu
from jax.experimental.pallas import tpu_sc as plsc

N_NODES = 100000
N_EDGES = 1600000
D = 32
L = 16
NC = 2
NS = 16
NW = NC * NS
SUB = 128
NSUB = 4
CHUNK = SUB * NSUB
NCHUNKS = N_EDGES // CHUNK
BASE_CHUNKS = NCHUNKS // NW
EXTRA = NCHUNKS - BASE_CHUNKS * NW
DUMP = 6256
DUMP_LAST = N_NODES - 15 * DUMP

_mesh = plsc.VectorSubcoreMesh(core_axis_name="c", subcore_axis_name="s")


def _edge_body(src2_h, dst2_h, eu_h, ev_h, g2b_h, zeros_h,
               ag_h, psum0_h, psum1_h,
               src_v, dst_v, u_v, v_v, ag_v, g2_v, sums_s,
               sem_u, sem_v):
    cid = lax.axis_index("c")
    sid = lax.axis_index("s")
    wid = sid * NC + cid

    @pl.when(sid == 0)
    def _():
        pltpu.sync_copy(zeros_h, sums_s)

    pltpu.sync_copy(g2b_h, g2_v)
    plsc.subcore_barrier()

    nmine = jnp.where(wid < EXTRA, BASE_CHUNKS + 1, BASE_CHUNKS)

    def chunk_body(k, carry):
        chunk = wid + k * NW
        row0 = chunk * NSUB
        ebase = chunk * CHUNK
        pltpu.sync_copy(src2_h.at[pl.ds(row0, NSUB)], src_v)
        pltpu.sync_copy(dst2_h.at[pl.ds(row0, NSUB)], dst_v)
        for j in range(NSUB):
            cu = pltpu.async_copy(eu_h.at[src_v.at[j]], u_v, sem_u)
            cv = pltpu.async_copy(ev_h.at[dst_v.at[j]], v_v, sem_v)
            cu.wait()
            cv.wait()

            def grp(kg, c2):
                eidx = kg * L + lax.iota(jnp.int32, L)
                acc = jnp.zeros((L,), jnp.float32)
                for d in range(D):
                    dsp = jnp.full((L,), d, jnp.int32)
                    ud = plsc.load_gather(u_v, [eidx, dsp])
                    vd = plsc.load_gather(v_v, [eidx, dsp])
                    acc = acc + ud * vd * g2_v[d]
                t = jnp.exp(acc)
                ag = 1.0 / (1.0 + jnp.exp(-t))
                ag_v[pl.ds(kg * L, L)] = ag
                return c2

            lax.fori_loop(0, SUB // L, grp, 0)
            pltpu.sync_copy(ag_v, sums_s.at[src_v.at[j]], add=True)
            pltpu.sync_copy(ag_v, ag_h.at[pl.ds(ebase + j * SUB, SUB)])
        return carry

    lax.fori_loop(0, nmine, chunk_body, 0)
    plsc.subcore_barrier()

    for c in range(NC):
        psum_h = (psum0_h, psum1_h)[c]

        @pl.when(jnp.logical_and(cid == c, sid < NS - 1))
        def _():
            pltpu.sync_copy(sums_s.at[pl.ds(sid * DUMP, DUMP)],
                            psum_h.at[pl.ds(sid * DUMP, DUMP)])

        @pl.when(jnp.logical_and(cid == c, sid == NS - 1))
        def _():
            pltpu.sync_copy(sums_s.at[pl.ds((NS - 1) * DUMP, DUMP_LAST)],
                            psum_h.at[pl.ds((NS - 1) * DUMP, DUMP_LAST)])


_edge_kernel = functools.partial(
    pl.kernel,
    out_type=(
        jax.ShapeDtypeStruct((N_EDGES,), jnp.float32),
        jax.ShapeDtypeStruct((N_NODES,), jnp.float32),
        jax.ShapeDtypeStruct((N_NODES,), jnp.float32),
    ),
    mesh=_mesh,
    scratch_types=[
        pltpu.VMEM((NSUB, SUB), jnp.int32),
        pltpu.VMEM((NSUB, SUB), jnp.int32),
        pltpu.VMEM((SUB, D), jnp.float32),
        pltpu.VMEM((SUB, D), jnp.float32),
        pltpu.VMEM((SUB,), jnp.float32),
        pltpu.VMEM((D, L), jnp.float32),
        pltpu.VMEM_SHARED((N_NODES,), jnp.float32),
        pltpu.SemaphoreType.DMA,
        pltpu.SemaphoreType.DMA,
    ],
    compiler_params=pltpu.CompilerParams(needs_layout_passes=False,
                                         use_tc_tiling_on_sc=False),
)(_edge_body)


def _norm_body(src2_h, ag_h, p0_h, p1_h,
               c_h,
               src_v, s0_v, s1_v, ag_v, c_v,
               sem0, sem1):
    cid = lax.axis_index("c")
    sid = lax.axis_index("s")
    wid = sid * NC + cid
    nmine = jnp.where(wid < EXTRA, BASE_CHUNKS + 1, BASE_CHUNKS)

    def chunk_body(k, carry):
        chunk = wid + k * NW
        row0 = chunk * NSUB
        ebase = chunk * CHUNK
        pltpu.sync_copy(src2_h.at[pl.ds(row0, NSUB)], src_v)
        for j in range(NSUB):
            c0 = pltpu.async_copy(p0_h.at[src_v.at[j]], s0_v, sem0)
            c1 = pltpu.async_copy(p1_h.at[src_v.at[j]], s1_v, sem1)
            pltpu.sync_copy(ag_h.at[pl.ds(ebase + j * SUB, SUB)], ag_v)
            c0.wait()
            c1.wait()

            def grp(kg, c2):
                sl = pl.ds(kg * L, L)
                ag = ag_v[sl]
                s = s0_v[sl] + s1_v[sl]
                c = jnp.minimum(jnp.maximum(ag * 5.0 / s, 0.0), 1.0)
                c_v[sl] = c
                return c2

            lax.fori_loop(0, SUB // L, grp, 0)
            pltpu.sync_copy(c_v, c_h.at[pl.ds(ebase + j * SUB, SUB)])
        return carry

    lax.fori_loop(0, nmine, chunk_body, 0)


_norm_kernel = functools.partial(
    pl.kernel,
    out_type=jax.ShapeDtypeStruct((N_EDGES,), jnp.float32),
    mesh=_mesh,
    scratch_types=[
        pltpu.VMEM((NSUB, SUB), jnp.int32),
        pltpu.VMEM((SUB,), jnp.float32),
        pltpu.VMEM((SUB,), jnp.float32),
        pltpu.VMEM((SUB,), jnp.float32),
        pltpu.VMEM((SUB,), jnp.float32),
        pltpu.SemaphoreType.DMA,
        pltpu.SemaphoreType.DMA,
    ],
)(_norm_body)


def kernel(Eu, Ev, edge_index, g):
    src = edge_index[0]
    dst = edge_index[1]
    src2 = src.reshape(N_EDGES // SUB, SUB)
    dst2 = dst.reshape(N_EDGES // SUB, SUB)
    g2 = (g * g).reshape(D)
    g2b = jnp.broadcast_to(g2[:, None], (D, L))
    zeros = jnp.zeros((N_NODES,), jnp.float32)
    ag, psum0, psum1 = _edge_kernel(src2, dst2, Eu, Ev, g2b, zeros)
    return _norm_kernel(src2, ag, psum0, psum1)


# speedup vs baseline: 9.9997x; 1.2413x over previous
import functools

import jax
import jax.numpy as jnp
from jax import lax
from jax.experimental import pallas as pl
from jax.experimental.pallas import tpu as pltpu
from jax.experimental.pallas import tpu_sc as plsc

N_NODES = 100000
N_EDGES = 1600000
D = 32
L = 16
NC = 2
NS = 16
NW = NC * NS
SUB = 128
NSUB = 4
CHUNK = SUB * NSUB
NCHUNKS = N_EDGES // CHUNK
BASE_CHUNKS = NCHUNKS // NW
EXTRA = NCHUNKS - BASE_CHUNKS * NW
DUMP = 6256
DUMP_LAST = N_NODES - 15 * DUMP

_mesh = plsc.VectorSubcoreMesh(core_axis_name="c", subcore_axis_name="s")


def _edge_body(src2_h, dst2_h, eu_h, ev_h, g2b_h, zeros_h,
               ag_h, psum0_h, psum1_h,
               src_v, dst_v, u_v, v_v, ag_v, g2_v, sums_s,
               sem_idx, sem_g0, sem_g1, sem_g2, sem_g3, sem_s):
    cid = lax.axis_index("c")
    sid = lax.axis_index("s")
    wid = sid * NC + cid
    gsems = (sem_g0, sem_g1, sem_g2, sem_g3)

    @pl.when(sid == 0)
    def _():
        pltpu.sync_copy(zeros_h, sums_s)

    pltpu.sync_copy(g2b_h, g2_v)
    plsc.subcore_barrier()

    nmine = jnp.where(wid < EXTRA, BASE_CHUNKS + 1, BASE_CHUNKS)

    def fire_idx(k, p):
        row0 = (wid + k * NW) * NSUB
        pltpu.async_copy(src2_h.at[pl.ds(row0, NSUB)], src_v.at[p], sem_idx)
        pltpu.async_copy(dst2_h.at[pl.ds(row0, NSUB)], dst_v.at[p], sem_idx)

    def drain_idx(p):
        pltpu.make_async_copy(src2_h.at[pl.ds(0, NSUB)], src_v.at[p],
                              sem_idx).wait()
        pltpu.make_async_copy(dst2_h.at[pl.ds(0, NSUB)], dst_v.at[p],
                              sem_idx).wait()

    fire_idx(0, 0)

    def chunk_body(k, carry):
        p = lax.rem(k, 2)
        chunk = wid + k * NW
        ebase = chunk * CHUNK
        drain_idx(p)

        @pl.when(k + 1 < nmine)
        def _():
            fire_idx(k + 1, 1 - p)

        copies = []
        for j in range(NSUB):
            cu = pltpu.async_copy(eu_h.at[src_v.at[p, j]], u_v.at[j],
                                  gsems[j])
            cv = pltpu.async_copy(ev_h.at[dst_v.at[p, j]], v_v.at[j],
                                  gsems[j])
            copies.append((cu, cv))

        outs = []
        for j in range(NSUB):
            cu, cv = copies[j]
            cu.wait()
            cv.wait()

            def grp(kg, c2, j=j):
                eidx = kg * L + lax.iota(jnp.int32, L)
                acc = jnp.zeros((L,), jnp.float32)
                for d in range(D):
                    dsp = jnp.full((L,), d, jnp.int32)
                    ud = plsc.load_gather(u_v.at[j], [eidx, dsp])
                    vd = plsc.load_gather(v_v.at[j], [eidx, dsp])
                    acc = acc + ud * vd * g2_v[d]
                t = jnp.exp(acc)
                ag = 1.0 / (1.0 + jnp.exp(-t))
                ag_v[j, pl.ds(kg * L, L)] = ag
                return c2

            lax.fori_loop(0, SUB // L, grp, 0)
            pltpu.sync_copy(ag_v.at[j], sums_s.at[src_v.at[p, j]], add=True)
            pltpu.sync_copy(ag_v.at[j], ag_h.at[pl.ds(ebase + j * SUB, SUB)])
        del outs
        return carry

    lax.fori_loop(0, nmine, chunk_body, 0)
    plsc.subcore_barrier()

    for c in range(NC):
        psum_h = (psum0_h, psum1_h)[c]

        @pl.when(jnp.logical_and(cid == c, sid < NS - 1))
        def _():
            pltpu.sync_copy(sums_s.at[pl.ds(sid * DUMP, DUMP)],
                            psum_h.at[pl.ds(sid * DUMP, DUMP)])

        @pl.when(jnp.logical_and(cid == c, sid == NS - 1))
        def _():
            pltpu.sync_copy(sums_s.at[pl.ds((NS - 1) * DUMP, DUMP_LAST)],
                            psum_h.at[pl.ds((NS - 1) * DUMP, DUMP_LAST)])


_edge_kernel = functools.partial(
    pl.kernel,
    out_type=(
        jax.ShapeDtypeStruct((N_EDGES,), jnp.float32),
        jax.ShapeDtypeStruct((N_NODES,), jnp.float32),
        jax.ShapeDtypeStruct((N_NODES,), jnp.float32),
    ),
    mesh=_mesh,
    scratch_types=[
        pltpu.VMEM((2, NSUB, SUB), jnp.int32),
        pltpu.VMEM((2, NSUB, SUB), jnp.int32),
        pltpu.VMEM((NSUB, SUB, D), jnp.float32),
        pltpu.VMEM((NSUB, SUB, D), jnp.float32),
        pltpu.VMEM((NSUB, SUB), jnp.float32),
        pltpu.VMEM((D, L), jnp.float32),
        pltpu.VMEM_SHARED((N_NODES,), jnp.float32),
        pltpu.SemaphoreType.DMA,
        pltpu.SemaphoreType.DMA,
        pltpu.SemaphoreType.DMA,
        pltpu.SemaphoreType.DMA,
        pltpu.SemaphoreType.DMA,
        pltpu.SemaphoreType.DMA,
    ],
    compiler_params=pltpu.CompilerParams(needs_layout_passes=False,
                                         use_tc_tiling_on_sc=False),
)(_edge_body)


def _norm_body(src2_h, ag_h, p0_h, p1_h,
               c_h,
               src_v, s0_v, s1_v, ag_v, c_v,
               sem_idx, sem_g0, sem_g1, sem_g2, sem_g3, sem_s):
    cid = lax.axis_index("c")
    sid = lax.axis_index("s")
    wid = sid * NC + cid
    gsems = (sem_g0, sem_g1, sem_g2, sem_g3)
    nmine = jnp.where(wid < EXTRA, BASE_CHUNKS + 1, BASE_CHUNKS)

    def fire_idx(k, p):
        row0 = (wid + k * NW) * NSUB
        pltpu.async_copy(src2_h.at[pl.ds(row0, NSUB)], src_v.at[p], sem_idx)

    def drain_idx(p):
        pltpu.make_async_copy(src2_h.at[pl.ds(0, NSUB)], src_v.at[p],
                              sem_idx).wait()

    fire_idx(0, 0)

    def chunk_body(k, carry):
        p = lax.rem(k, 2)
        chunk = wid + k * NW
        ebase = chunk * CHUNK
        drain_idx(p)

        @pl.when(k + 1 < nmine)
        def _():
            fire_idx(k + 1, 1 - p)

        copies = []
        for j in range(NSUB):
            c0 = pltpu.async_copy(p0_h.at[src_v.at[p, j]], s0_v.at[j],
                                  gsems[j])
            c1 = pltpu.async_copy(p1_h.at[src_v.at[p, j]], s1_v.at[j],
                                  gsems[j])
            ca = pltpu.async_copy(ag_h.at[pl.ds(ebase + j * SUB, SUB)],
                                  ag_v.at[j], gsems[j])
            copies.append((c0, c1, ca))

        outs = []
        for j in range(NSUB):
            for c in copies[j]:
                c.wait()

            def grp(kg, c2, j=j):
                sl = pl.ds(kg * L, L)
                ag = ag_v[j, sl]
                s = s0_v[j, sl] + s1_v[j, sl]
                c_v[j, sl] = jnp.minimum(jnp.maximum(ag * 5.0 / s, 0.0), 1.0)
                return c2

            lax.fori_loop(0, SUB // L, grp, 0)
            outs.append(pltpu.async_copy(
                c_v.at[j], c_h.at[pl.ds(ebase + j * SUB, SUB)], sem_s))
        for c in outs:
            c.wait()
        return carry

    lax.fori_loop(0, nmine, chunk_body, 0)


_norm_kernel = functools.partial(
    pl.kernel,
    out_type=jax.ShapeDtypeStruct((N_EDGES,), jnp.float32),
    mesh=_mesh,
    scratch_types=[
        pltpu.VMEM((2, NSUB, SUB), jnp.int32),
        pltpu.VMEM((NSUB, SUB), jnp.float32),
        pltpu.VMEM((NSUB, SUB), jnp.float32),
        pltpu.VMEM((NSUB, SUB), jnp.float32),
        pltpu.VMEM((NSUB, SUB), jnp.float32),
        pltpu.SemaphoreType.DMA,
        pltpu.SemaphoreType.DMA,
        pltpu.SemaphoreType.DMA,
        pltpu.SemaphoreType.DMA,
        pltpu.SemaphoreType.DMA,
        pltpu.SemaphoreType.DMA,
    ],
    compiler_params=pltpu.CompilerParams(needs_layout_passes=False,
                                         use_tc_tiling_on_sc=False),
)(_norm_body)


def kernel(Eu, Ev, edge_index, g):
    src = edge_index[0]
    dst = edge_index[1]
    src2 = src.reshape(N_EDGES // SUB, SUB)
    dst2 = dst.reshape(N_EDGES // SUB, SUB)
    g2 = (g * g).reshape(D)
    g2b = jnp.broadcast_to(g2[:, None], (D, L))
    zeros = jnp.zeros((N_NODES,), jnp.float32)
    ag, psum0, psum1 = _edge_kernel(src2, dst2, Eu, Ev, g2b, zeros)
    return _norm_kernel(src2, ag, psum0, psum1)


# speedup vs baseline: 10.2522x; 1.0253x over previous
import functools

import jax
import jax.numpy as jnp
from jax import lax
from jax.experimental import pallas as pl
from jax.experimental.pallas import tpu as pltpu
from jax.experimental.pallas import tpu_sc as plsc

N_NODES = 100000
N_EDGES = 1600000
D = 32
L = 16
NC = 2
NS = 16
NW = NC * NS
SUB = 128
NSUB = 4
CHUNK = SUB * NSUB
NCHUNKS = N_EDGES // CHUNK
BASE_CHUNKS = NCHUNKS // NW
EXTRA = NCHUNKS - BASE_CHUNKS * NW
DUMP = 6256
DUMP_LAST = N_NODES - 15 * DUMP

_mesh = plsc.VectorSubcoreMesh(core_axis_name="c", subcore_axis_name="s")


def _edge_body(src2_h, dst2_h, eu_h, ev_h, g2b_h, zeros_h,
               ag_h, psum0_h, psum1_h,
               src_v, dst_v, u_v, v_v, ag_v, g2_v, sums_s,
               sem_idx, sem_g0, sem_g1, sem_g2, sem_g3, sem_s, sem_a):
    cid = lax.axis_index("c")
    sid = lax.axis_index("s")
    wid = sid * NC + cid
    gsems = (sem_g0, sem_g1, sem_g2, sem_g3)

    @pl.when(sid == 0)
    def _():
        pltpu.sync_copy(zeros_h, sums_s)

    pltpu.sync_copy(g2b_h, g2_v)
    plsc.subcore_barrier()

    nmine = jnp.where(wid < EXTRA, BASE_CHUNKS + 1, BASE_CHUNKS)

    def fire_idx(k, p):
        row0 = (wid + k * NW) * NSUB
        pltpu.async_copy(src2_h.at[pl.ds(row0, NSUB)], src_v.at[p], sem_idx)
        pltpu.async_copy(dst2_h.at[pl.ds(row0, NSUB)], dst_v.at[p], sem_idx)

    def drain_idx(p):
        pltpu.make_async_copy(src2_h.at[pl.ds(0, NSUB)], src_v.at[p],
                              sem_idx).wait()
        pltpu.make_async_copy(dst2_h.at[pl.ds(0, NSUB)], dst_v.at[p],
                              sem_idx).wait()

    fire_idx(0, 0)

    def chunk_body(k, carry):
        p = lax.rem(k, 2)
        chunk = wid + k * NW
        ebase = chunk * CHUNK
        drain_idx(p)

        @pl.when(k + 1 < nmine)
        def _():
            fire_idx(k + 1, 1 - p)

        copies = []
        for j in range(NSUB):
            cu = pltpu.async_copy(eu_h.at[src_v.at[p, j]], u_v.at[j],
                                  gsems[j])
            cv = pltpu.async_copy(ev_h.at[dst_v.at[p, j]], v_v.at[j],
                                  gsems[j])
            copies.append((cu, cv))

        outs = []
        for j in range(NSUB):
            cu, cv = copies[j]
            cu.wait()
            cv.wait()

            def grp(kg, c2, j=j):
                eidx = kg * L + lax.iota(jnp.int32, L)
                acc = jnp.zeros((L,), jnp.float32)
                for d in range(D):
                    dsp = jnp.full((L,), d, jnp.int32)
                    ud = plsc.load_gather(u_v.at[j], [eidx, dsp])
                    vd = plsc.load_gather(v_v.at[j], [eidx, dsp])
                    acc = acc + ud * vd * g2_v[d]
                t = jnp.exp(acc)
                ag = 1.0 / (1.0 + jnp.exp(-t))
                ag_v[j, pl.ds(kg * L, L)] = ag
                return c2

            lax.fori_loop(0, SUB // L, grp, 0)
            outs.append(pltpu.async_copy(
                ag_v.at[j], sums_s.at[src_v.at[p, j]], sem_a, add=True))
            outs.append(pltpu.async_copy(
                ag_v.at[j], ag_h.at[pl.ds(ebase + j * SUB, SUB)], sem_s))
        for c in outs:
            c.wait()
        return carry

    lax.fori_loop(0, nmine, chunk_body, 0)
    plsc.subcore_barrier()

    for c in range(NC):
        psum_h = (psum0_h, psum1_h)[c]

        @pl.when(jnp.logical_and(cid == c, sid < NS - 1))
        def _():
            pltpu.sync_copy(sums_s.at[pl.ds(sid * DUMP, DUMP)],
                            psum_h.at[pl.ds(sid * DUMP, DUMP)])

        @pl.when(jnp.logical_and(cid == c, sid == NS - 1))
        def _():
            pltpu.sync_copy(sums_s.at[pl.ds((NS - 1) * DUMP, DUMP_LAST)],
                            psum_h.at[pl.ds((NS - 1) * DUMP, DUMP_LAST)])


_edge_kernel = functools.partial(
    pl.kernel,
    out_type=(
        jax.ShapeDtypeStruct((N_EDGES,), jnp.float32),
        jax.ShapeDtypeStruct((N_NODES,), jnp.float32),
        jax.ShapeDtypeStruct((N_NODES,), jnp.float32),
    ),
    mesh=_mesh,
    scratch_types=[
        pltpu.VMEM((2, NSUB, SUB), jnp.int32),
        pltpu.VMEM((2, NSUB, SUB), jnp.int32),
        pltpu.VMEM((NSUB, SUB, D), jnp.float32),
        pltpu.VMEM((NSUB, SUB, D), jnp.float32),
        pltpu.VMEM((NSUB, SUB), jnp.float32),
        pltpu.VMEM((D, L), jnp.float32),
        pltpu.VMEM_SHARED((N_NODES,), jnp.float32),
        pltpu.SemaphoreType.DMA,
        pltpu.SemaphoreType.DMA,
        pltpu.SemaphoreType.DMA,
        pltpu.SemaphoreType.DMA,
        pltpu.SemaphoreType.DMA,
        pltpu.SemaphoreType.DMA,
        pltpu.SemaphoreType.DMA,
    ],
    compiler_params=pltpu.CompilerParams(needs_layout_passes=False,
                                         use_tc_tiling_on_sc=False),
)(_edge_body)


def _norm_body(src2_h, ag_h, p0_h, p1_h,
               c_h,
               src_v, s0_v, s1_v, ag_v, c_v,
               sem_idx, sem_g0, sem_g1, sem_g2, sem_g3, sem_s):
    cid = lax.axis_index("c")
    sid = lax.axis_index("s")
    wid = sid * NC + cid
    gsems = (sem_g0, sem_g1, sem_g2, sem_g3)
    nmine = jnp.where(wid < EXTRA, BASE_CHUNKS + 1, BASE_CHUNKS)

    def fire_idx(k, p):
        row0 = (wid + k * NW) * NSUB
        pltpu.async_copy(src2_h.at[pl.ds(row0, NSUB)], src_v.at[p], sem_idx)

    def drain_idx(p):
        pltpu.make_async_copy(src2_h.at[pl.ds(0, NSUB)], src_v.at[p],
                              sem_idx).wait()

    fire_idx(0, 0)

    def chunk_body(k, carry):
        p = lax.rem(k, 2)
        chunk = wid + k * NW
        ebase = chunk * CHUNK
        drain_idx(p)

        @pl.when(k + 1 < nmine)
        def _():
            fire_idx(k + 1, 1 - p)

        copies = []
        for j in range(NSUB):
            c0 = pltpu.async_copy(p0_h.at[src_v.at[p, j]], s0_v.at[j],
                                  gsems[j])
            c1 = pltpu.async_copy(p1_h.at[src_v.at[p, j]], s1_v.at[j],
                                  gsems[j])
            ca = pltpu.async_copy(ag_h.at[pl.ds(ebase + j * SUB, SUB)],
                                  ag_v.at[j], gsems[j])
            copies.append((c0, c1, ca))

        outs = []
        for j in range(NSUB):
            for c in copies[j]:
                c.wait()

            def grp(kg, c2, j=j):
                sl = pl.ds(kg * L, L)
                ag = ag_v[j, sl]
                s = s0_v[j, sl] + s1_v[j, sl]
                c_v[j, sl] = jnp.minimum(jnp.maximum(ag * 5.0 / s, 0.0), 1.0)
                return c2

            lax.fori_loop(0, SUB // L, grp, 0)
            outs.append(pltpu.async_copy(
                c_v.at[j], c_h.at[pl.ds(ebase + j * SUB, SUB)], sem_s))
        for c in outs:
            c.wait()
        return carry

    lax.fori_loop(0, nmine, chunk_body, 0)


_norm_kernel = functools.partial(
    pl.kernel,
    out_type=jax.ShapeDtypeStruct((N_EDGES,), jnp.float32),
    mesh=_mesh,
    scratch_types=[
        pltpu.VMEM((2, NSUB, SUB), jnp.int32),
        pltpu.VMEM((NSUB, SUB), jnp.float32),
        pltpu.VMEM((NSUB, SUB), jnp.float32),
        pltpu.VMEM((NSUB, SUB), jnp.float32),
        pltpu.VMEM((NSUB, SUB), jnp.float32),
        pltpu.SemaphoreType.DMA,
        pltpu.SemaphoreType.DMA,
        pltpu.SemaphoreType.DMA,
        pltpu.SemaphoreType.DMA,
        pltpu.SemaphoreType.DMA,
        pltpu.SemaphoreType.DMA,
    ],
    compiler_params=pltpu.CompilerParams(needs_layout_passes=False,
                                         use_tc_tiling_on_sc=False),
)(_norm_body)


def kernel(Eu, Ev, edge_index, g):
    src = edge_index[0]
    dst = edge_index[1]
    src2 = src.reshape(N_EDGES // SUB, SUB)
    dst2 = dst.reshape(N_EDGES // SUB, SUB)
    g2 = (g * g).reshape(D)
    g2b = jnp.broadcast_to(g2[:, None], (D, L))
    zeros = jnp.zeros((N_NODES,), jnp.float32)
    ag, psum0, psum1 = _edge_kernel(src2, dst2, Eu, Ev, g2b, zeros)
    return _norm_kernel(src2, ag, psum0, psum1)
